# baseline (speedup 1.0000x reference)
import functools

import jax
import jax.numpy as jnp
from jax import lax
from jax.experimental import pallas as pl
from jax.experimental.pallas import tpu as pltpu
from jax.experimental.pallas import tpu_sc as plsc

N = 10000
E = 320000
D = 128
DH = D // 2

NC = 2
NS = 16
NW = NC * NS
K = 128
CH = 80
EPAD = NW * CH * K
NPAD = 10240
RPT = NPAD // NS
NCHK = RPT // K

_sc_mesh = plsc.VectorSubcoreMesh(core_axis_name="c", subcore_axis_name="s")
_sc_params = pltpu.CompilerParams(use_tc_tiling_on_sc=False)


def _zero_rows(ref, nrows, ncols):
    def body(i):
        for c in range(ncols // 16):
            ref[i, pl.ds(c * 16, 16)] = jnp.zeros((16,), jnp.float32)
    lax.fori_loop(0, nrows, lambda i, _: (body(i), 0)[1], 0)


@functools.partial(
    pl.kernel,
    out_type=jax.ShapeDtypeStruct((NC, 2, NPAD), jnp.float32),
    mesh=_sc_mesh,
    compiler_params=_sc_params,
    scratch_types=[
        pltpu.VMEM((CH, K), jnp.int32),
        pltpu.VMEM((CH, K), jnp.int32),
        pltpu.VMEM((K,), jnp.float32),
        pltpu.VMEM((RPT,), jnp.float32),
        pltpu.VMEM_SHARED((NPAD,), jnp.float32),
        pltpu.VMEM_SHARED((NPAD,), jnp.float32),
        pltpu.SemaphoreType.DMA,
        pltpu.SemaphoreType.DMA,
    ],
)
def _count_kernel(src_hbm, dst_hbm, out_hbm, src_v, dst_v, ones_v, zeros_v,
                  cs_sh, cd_sh, sema, semb):
    cid = lax.axis_index("c")
    sid = lax.axis_index("s")
    w = cid * NS + sid

    pltpu.sync_copy(src_hbm.at[pl.ds(w * CH, CH)], src_v)
    pltpu.sync_copy(dst_hbm.at[pl.ds(w * CH, CH)], dst_v)

    def fill(i, _):
        ones_v[pl.ds(i * 16, 16)] = jnp.ones((16,), jnp.float32)
        return 0
    lax.fori_loop(0, K // 16, fill, 0)

    def fillz(i, _):
        zeros_v[pl.ds(i * 16, 16)] = jnp.zeros((16,), jnp.float32)
        return 0
    lax.fori_loop(0, RPT // 16, fillz, 0)

    base = sid * RPT
    pltpu.sync_copy(zeros_v, cs_sh.at[pl.ds(base, RPT)])
    pltpu.sync_copy(zeros_v, cd_sh.at[pl.ds(base, RPT)])
    plsc.subcore_barrier()

    def fire(i):
        def go(b):
            j = 8 * i + b
            pltpu.async_copy(ones_v, cs_sh.at[src_v.at[j]], sema, add=True)
            pltpu.async_copy(ones_v, cd_sh.at[dst_v.at[j]], semb, add=True)
        for b in range(8):
            go(b)

    def drain(i):
        def go(b):
            j = 8 * i + b
            pltpu.make_async_copy(ones_v, cs_sh.at[src_v.at[j]], sema).wait()
            pltpu.make_async_copy(ones_v, cd_sh.at[dst_v.at[j]], semb).wait()
        for b in range(8):
            go(b)

    fire(0)

    def body(i, _):
        @pl.when(i + 1 < CH // 8)
        def _():
            fire(i + 1)
        drain(i)
        return 0
    lax.fori_loop(0, CH // 8, body, 0)
    plsc.subcore_barrier()

    pltpu.sync_copy(cs_sh.at[pl.ds(base, RPT)], out_hbm.at[cid, 0, pl.ds(base, RPT)])
    pltpu.sync_copy(cd_sh.at[pl.ds(base, RPT)], out_hbm.at[cid, 1, pl.ds(base, RPT)])


CPT = NW * CH // NS
NBUF = 5
AHEAD = 4


@functools.partial(
    pl.kernel,
    out_type=jax.ShapeDtypeStruct((NPAD, D), jnp.float32),
    mesh=_sc_mesh,
    compiler_params=_sc_params,
    scratch_types=[
        pltpu.VMEM((CPT, K), jnp.int32),
        pltpu.VMEM((CPT, K), jnp.int32),
        pltpu.VMEM((NBUF, K, DH), jnp.float32),
        pltpu.VMEM_SHARED((NPAD, DH), jnp.float32),
        [pltpu.SemaphoreType.DMA] * NBUF,
        [pltpu.SemaphoreType.DMA] * NBUF,
    ],
)
def _agg_kernel(u_hbm, src_hbm, dst_hbm, out_hbm, src_v, dst_v, rows_v,
                acc_sh, gsems, ssems):
    cid = lax.axis_index("c")
    sid = lax.axis_index("s")

    pltpu.sync_copy(src_hbm.at[pl.ds(sid * CPT, CPT)], src_v)
    pltpu.sync_copy(dst_hbm.at[pl.ds(sid * CPT, CPT)], dst_v)

    def scale(j, _):
        for c in range(K // 16):
            sl = pl.ds(c * 16, 16)
            src_v[j, sl] = src_v[j, sl] * 2 + cid
        return 0
    lax.fori_loop(0, CPT, scale, 0)

    _zero_rows(rows_v.at[0], K, DH)
    for t in range(NCHK):
        base = sid * RPT + t * K
        pltpu.make_async_copy(rows_v.at[0], acc_sh.at[pl.ds(base, K)],
                              gsems[0]).start()
    for t in range(NCHK):
        base = sid * RPT + t * K
        pltpu.make_async_copy(rows_v.at[0], acc_sh.at[pl.ds(base, K)],
                              gsems[0]).wait()
    plsc.subcore_barrier()

    def gather(j, b):
        return pltpu.make_async_copy(u_hbm.at[src_v.at[j]], rows_v.at[b],
                                     gsems[b])

    def scatter_wait(j, b):
        pltpu.make_async_copy(rows_v.at[b], acc_sh.at[dst_v.at[j]],
                              ssems[b]).wait()

    for b in range(AHEAD):
        gather(b, b).start()

    LAG = NBUF - AHEAD

    def body(i, _):
        for b in range(NBUF):
            j = NBUF * i + b
            gather(j, b).wait()
            bn = (b + AHEAD) % NBUF

            @pl.when((j >= LAG) & (j + AHEAD < CPT))
            def _():
                scatter_wait(j - LAG, bn)

            @pl.when(j + AHEAD < CPT)
            def _():
                gather(j + AHEAD, bn).start()

            pltpu.async_copy(rows_v.at[b], acc_sh.at[dst_v.at[j]],
                             ssems[b], add=True)
        return 0
    lax.fori_loop(0, CPT // NBUF, body, 0)

    for b in range(NBUF):
        scatter_wait(CPT - NBUF + b, (CPT - NBUF + b) % NBUF)
    plsc.subcore_barrier()

    for t in range(NCHK):
        base = sid * RPT + t * K
        pltpu.make_async_copy(
            acc_sh.at[pl.ds(base, K)],
            out_hbm.at[pl.ds(base, K), pl.ds(cid * DH, DH)], gsems[0]).start()
    for t in range(NCHK):
        base = sid * RPT + t * K
        pltpu.make_async_copy(
            acc_sh.at[pl.ds(base, K)],
            out_hbm.at[pl.ds(base, K), pl.ds(cid * DH, DH)], gsems[0]).wait()


def _mm0_body(x_ref, w_ref, cs_ref, u_ref):
    ns = lax.rsqrt(jnp.maximum(cs_ref[0] + cs_ref[1], 1.0))
    h = x_ref[...] * ns
    u_ref[...] = jnp.dot(h, w_ref[...], preferred_element_type=jnp.float32)


def _finmm_body(agg_ref, b_ref, wn_ref, cs_ref, cd_ref, z_ref, un_ref):
    nd = lax.rsqrt(jnp.maximum(cd_ref[0] + cd_ref[1], 1.0))
    ns = lax.rsqrt(jnp.maximum(cs_ref[0] + cs_ref[1], 1.0))
    z = agg_ref[...] * nd + b_ref[...]
    z_ref[...] = z
    xn = jnp.maximum(z, 0.0) * ns
    un_ref[...] = jnp.dot(xn, wn_ref[...], preferred_element_type=jnp.float32)


_GB = 8
_BLK = NPAD // _GB

_row_spec = pl.BlockSpec((_BLK, D), lambda i: (i, 0))
_w_spec = pl.BlockSpec((D, D), lambda i: (0, 0))
_b_spec = pl.BlockSpec((1, D), lambda i: (0, 0))
_cnt_spec = pl.BlockSpec((NC, _BLK, 1), lambda i: (0, i, 0))

_mm0 = pl.pallas_call(
    _mm0_body,
    grid=(_GB,),
    in_specs=[_row_spec, _w_spec, _cnt_spec],
    out_specs=_row_spec,
    out_shape=jax.ShapeDtypeStruct((NPAD, D), jnp.float32),
)

_finmm = pl.pallas_call(
    _finmm_body,
    grid=(_GB,),
    in_specs=[_row_spec, _b_spec, _w_spec, _cnt_spec, _cnt_spec],
    out_specs=(_row_spec, _row_spec),
    out_shape=(
        jax.ShapeDtypeStruct((NPAD, D), jnp.float32),
        jax.ShapeDtypeStruct((NPAD, D), jnp.float32),
    ),
)


def kernel(features, edge_index, W1, b1, W2, b2):
    src = edge_index[0].astype(jnp.int32)
    dst = edge_index[1].astype(jnp.int32)
    pad_idx = N + (jnp.arange(EPAD - E, dtype=jnp.int32) % (NPAD - N))
    src_p = jnp.concatenate([src, pad_idx]).reshape(NW * CH, K)
    dst_p = jnp.concatenate([dst, pad_idx]).reshape(NW * CH, K)
    feats_p = jnp.pad(features, ((0, NPAD - N), (0, 0)))

    cnt = _count_kernel(src_p, dst_p)
    cs = cnt[:, 0, :, None]
    cd = cnt[:, 1, :, None]

    Ws = jnp.stack([W2, W2])
    bs = jnp.stack([b1.reshape(1, D), b2.reshape(1, D)])

    u1 = _mm0(feats_p, W1, cs)

    def layer(l, carry):
        u, _ = carry
        b = lax.dynamic_index_in_dim(bs, l, keepdims=False)
        wn = lax.dynamic_index_in_dim(Ws, l, keepdims=False)
        u_r = u.reshape(2 * NPAD, DH)
        agg = _agg_kernel(u_r, src_p, dst_p)
        z, un = _finmm(agg, b, wn, cs, cd)
        return (un, z)

    _, z = lax.fori_loop(0, 2, layer, (u1, feats_p))
    return z[:N]


# speedup vs baseline: 1.0097x; 1.0097x over previous
import functools

import jax
import jax.numpy as jnp
from jax import lax
from jax.experimental import pallas as pl
from jax.experimental.pallas import tpu as pltpu
from jax.experimental.pallas import tpu_sc as plsc

N = 10000
E = 320000
D = 128
DH = D // 2

NC = 2
NS = 16
NW = NC * NS
K = 128
CH = 80
EPAD = NW * CH * K
NPAD = 10240
RPT = NPAD // NS
NCHK = RPT // K

_sc_mesh = plsc.VectorSubcoreMesh(core_axis_name="c", subcore_axis_name="s")
_sc_params = pltpu.CompilerParams(use_tc_tiling_on_sc=False)


def _zero_rows(ref, nrows, ncols):
    def body(i):
        for c in range(ncols // 16):
            ref[i, pl.ds(c * 16, 16)] = jnp.zeros((16,), jnp.float32)
    lax.fori_loop(0, nrows, lambda i, _: (body(i), 0)[1], 0)


@functools.partial(
    pl.kernel,
    out_type=jax.ShapeDtypeStruct((NC, 2, NPAD), jnp.float32),
    mesh=_sc_mesh,
    compiler_params=_sc_params,
    scratch_types=[
        pltpu.VMEM((CH, K), jnp.int32),
        pltpu.VMEM((CH, K), jnp.int32),
        pltpu.VMEM((K,), jnp.float32),
        pltpu.VMEM((RPT,), jnp.float32),
        pltpu.VMEM_SHARED((NPAD,), jnp.float32),
        pltpu.VMEM_SHARED((NPAD,), jnp.float32),
        pltpu.SemaphoreType.DMA,
        pltpu.SemaphoreType.DMA,
    ],
)
def _count_kernel(src_hbm, dst_hbm, out_hbm, src_v, dst_v, ones_v, zeros_v,
                  cs_sh, cd_sh, sema, semb):
    cid = lax.axis_index("c")
    sid = lax.axis_index("s")
    w = cid * NS + sid

    pltpu.sync_copy(src_hbm.at[pl.ds(w * CH, CH)], src_v)
    pltpu.sync_copy(dst_hbm.at[pl.ds(w * CH, CH)], dst_v)

    def fill(i, _):
        ones_v[pl.ds(i * 16, 16)] = jnp.ones((16,), jnp.float32)
        return 0
    lax.fori_loop(0, K // 16, fill, 0)

    def fillz(i, _):
        zeros_v[pl.ds(i * 16, 16)] = jnp.zeros((16,), jnp.float32)
        return 0
    lax.fori_loop(0, RPT // 16, fillz, 0)

    base = sid * RPT
    pltpu.sync_copy(zeros_v, cs_sh.at[pl.ds(base, RPT)])
    pltpu.sync_copy(zeros_v, cd_sh.at[pl.ds(base, RPT)])
    plsc.subcore_barrier()

    def fire(i):
        def go(b):
            j = 8 * i + b
            pltpu.async_copy(ones_v, cs_sh.at[src_v.at[j]], sema, add=True)
            pltpu.async_copy(ones_v, cd_sh.at[dst_v.at[j]], semb, add=True)
        for b in range(8):
            go(b)

    def drain(i):
        def go(b):
            j = 8 * i + b
            pltpu.make_async_copy(ones_v, cs_sh.at[src_v.at[j]], sema).wait()
            pltpu.make_async_copy(ones_v, cd_sh.at[dst_v.at[j]], semb).wait()
        for b in range(8):
            go(b)

    fire(0)

    def body(i, _):
        @pl.when(i + 1 < CH // 8)
        def _():
            fire(i + 1)
        drain(i)
        return 0
    lax.fori_loop(0, CH // 8, body, 0)
    plsc.subcore_barrier()

    pltpu.sync_copy(cs_sh.at[pl.ds(base, RPT)], out_hbm.at[cid, 0, pl.ds(base, RPT)])
    pltpu.sync_copy(cd_sh.at[pl.ds(base, RPT)], out_hbm.at[cid, 1, pl.ds(base, RPT)])


CPT = NW * CH // NS
NBUF = 5
AHEAD = 4


@functools.partial(
    pl.kernel,
    out_type=jax.ShapeDtypeStruct((NPAD, D), jnp.float32),
    mesh=_sc_mesh,
    compiler_params=_sc_params,
    scratch_types=[
        pltpu.VMEM((CPT, K), jnp.int32),
        pltpu.VMEM((CPT, K), jnp.int32),
        pltpu.VMEM((NBUF, K, DH), jnp.float32),
        pltpu.VMEM_SHARED((NPAD, DH), jnp.float32),
        [pltpu.SemaphoreType.DMA] * NBUF,
        [pltpu.SemaphoreType.DMA] * NBUF,
    ],
)
def _agg_kernel(u_hbm, src_hbm, dst_hbm, out_hbm, src_v, dst_v, rows_v,
                acc_sh, gsems, ssems):
    cid = lax.axis_index("c")
    sid = lax.axis_index("s")

    pltpu.sync_copy(src_hbm.at[pl.ds(sid * CPT, CPT)], src_v)
    pltpu.sync_copy(dst_hbm.at[pl.ds(sid * CPT, CPT)], dst_v)

    def scale(j, _):
        for c in range(K // 16):
            sl = pl.ds(c * 16, 16)
            src_v[j, sl] = src_v[j, sl] * 2 + cid
        return 0
    lax.fori_loop(0, CPT, scale, 0)

    _zero_rows(rows_v.at[0], K, DH)
    for t in range(NCHK):
        base = sid * RPT + t * K
        pltpu.make_async_copy(rows_v.at[0], acc_sh.at[pl.ds(base, K)],
                              gsems[0]).start()
    for t in range(NCHK):
        base = sid * RPT + t * K
        pltpu.make_async_copy(rows_v.at[0], acc_sh.at[pl.ds(base, K)],
                              gsems[0]).wait()
    plsc.subcore_barrier()

    def gather(j, b):
        return pltpu.make_async_copy(u_hbm.at[src_v.at[j]], rows_v.at[b],
                                     gsems[b])

    def scatter_wait(j, b):
        pltpu.make_async_copy(rows_v.at[b], acc_sh.at[dst_v.at[j]],
                              ssems[b]).wait()

    for b in range(AHEAD):
        gather(b, b).start()

    LAG = NBUF - AHEAD

    def body(i, _):
        for b in range(NBUF):
            j = NBUF * i + b
            gather(j, b).wait()
            bn = (b + AHEAD) % NBUF

            @pl.when((j >= LAG) & (j + AHEAD < CPT))
            def _():
                scatter_wait(j - LAG, bn)

            @pl.when(j + AHEAD < CPT)
            def _():
                gather(j + AHEAD, bn).start()

            pltpu.async_copy(rows_v.at[b], acc_sh.at[dst_v.at[j]],
                             ssems[b], add=True)
        return 0
    lax.fori_loop(0, CPT // NBUF, body, 0)

    for b in range(NBUF):
        scatter_wait(CPT - NBUF + b, (CPT - NBUF + b) % NBUF)
    plsc.subcore_barrier()

    for t in range(NCHK):
        base = sid * RPT + t * K
        pltpu.make_async_copy(
            acc_sh.at[pl.ds(base, K)],
            out_hbm.at[pl.ds(base, K), pl.ds(cid * DH, DH)], gsems[0]).start()
    for t in range(NCHK):
        base = sid * RPT + t * K
        pltpu.make_async_copy(
            acc_sh.at[pl.ds(base, K)],
            out_hbm.at[pl.ds(base, K), pl.ds(cid * DH, DH)], gsems[0]).wait()


def _mm0_body(x_ref, w_ref, cs_ref, u_ref):
    ns = lax.rsqrt(jnp.maximum(cs_ref[0] + cs_ref[1], 1.0))
    h = x_ref[...] * ns
    u_ref[...] = jnp.dot(h, w_ref[...], preferred_element_type=jnp.float32)


def _finmm_body(agg_ref, b_ref, wn_ref, cs_ref, cd_ref, z_ref, un_ref):
    nd = lax.rsqrt(jnp.maximum(cd_ref[0] + cd_ref[1], 1.0))
    ns = lax.rsqrt(jnp.maximum(cs_ref[0] + cs_ref[1], 1.0))
    z = agg_ref[...] * nd + b_ref[...]
    z_ref[...] = z
    xn = jnp.maximum(z, 0.0) * ns
    un_ref[...] = jnp.dot(xn, wn_ref[...], preferred_element_type=jnp.float32)


_mm0 = pl.pallas_call(
    _mm0_body,
    out_shape=jax.ShapeDtypeStruct((NPAD, D), jnp.float32),
)

_finmm = pl.pallas_call(
    _finmm_body,
    out_shape=(
        jax.ShapeDtypeStruct((NPAD, D), jnp.float32),
        jax.ShapeDtypeStruct((NPAD, D), jnp.float32),
    ),
)


def kernel(features, edge_index, W1, b1, W2, b2):
    src = edge_index[0].astype(jnp.int32)
    dst = edge_index[1].astype(jnp.int32)
    pad_idx = N + (jnp.arange(EPAD - E, dtype=jnp.int32) % (NPAD - N))
    src_p = jnp.concatenate([src, pad_idx]).reshape(NW * CH, K)
    dst_p = jnp.concatenate([dst, pad_idx]).reshape(NW * CH, K)
    feats_p = jnp.pad(features, ((0, NPAD - N), (0, 0)))

    cnt = _count_kernel(src_p, dst_p)
    cs = cnt[:, 0, :, None]
    cd = cnt[:, 1, :, None]

    Ws = jnp.stack([W2, W2])
    bs = jnp.stack([b1.reshape(1, D), b2.reshape(1, D)])

    u1 = _mm0(feats_p, W1, cs)

    def layer(l, carry):
        u, _ = carry
        b = lax.dynamic_index_in_dim(bs, l, keepdims=False)
        wn = lax.dynamic_index_in_dim(Ws, l, keepdims=False)
        u_r = u.reshape(2 * NPAD, DH)
        agg = _agg_kernel(u_r, src_p, dst_p)
        z, un = _finmm(agg, b, wn, cs, cd)
        return (un, z)

    _, z = lax.fori_loop(0, 2, layer, (u1, feats_p))
    return z[:N]


# speedup vs baseline: 1.0361x; 1.0262x over previous
import functools

import jax
import jax.numpy as jnp
from jax import lax
from jax.experimental import pallas as pl
from jax.experimental.pallas import tpu as pltpu
from jax.experimental.pallas import tpu_sc as plsc

N = 10000
E = 320000
D = 128
DH = D // 2

NC = 2
NS = 16
NW = NC * NS
K = 128
CH = 80
EPAD = NW * CH * K
NPAD = 10240
RPT = NPAD // NS
NCHK = RPT // K

_sc_mesh = plsc.VectorSubcoreMesh(core_axis_name="c", subcore_axis_name="s")
_sc_params = pltpu.CompilerParams(use_tc_tiling_on_sc=False)


def _zero_rows(ref, nrows, ncols):
    def body(i):
        for c in range(ncols // 16):
            ref[i, pl.ds(c * 16, 16)] = jnp.zeros((16,), jnp.float32)
    lax.fori_loop(0, nrows, lambda i, _: (body(i), 0)[1], 0)


@functools.partial(
    pl.kernel,
    out_type=jax.ShapeDtypeStruct((NC, 2, NPAD), jnp.float32),
    mesh=_sc_mesh,
    compiler_params=_sc_params,
    scratch_types=[
        pltpu.VMEM((CH, K), jnp.int32),
        pltpu.VMEM((CH, K), jnp.int32),
        pltpu.VMEM((K,), jnp.float32),
        pltpu.VMEM((RPT,), jnp.float32),
        pltpu.VMEM_SHARED((NPAD,), jnp.float32),
        pltpu.VMEM_SHARED((NPAD,), jnp.float32),
        pltpu.SemaphoreType.DMA,
        pltpu.SemaphoreType.DMA,
    ],
)
def _count_kernel(src_hbm, dst_hbm, out_hbm, src_v, dst_v, ones_v, zeros_v,
                  cs_sh, cd_sh, sema, semb):
    cid = lax.axis_index("c")
    sid = lax.axis_index("s")
    w = cid * NS + sid

    pltpu.sync_copy(src_hbm.at[pl.ds(w * CH, CH)], src_v)
    pltpu.sync_copy(dst_hbm.at[pl.ds(w * CH, CH)], dst_v)

    def fill(i, _):
        ones_v[pl.ds(i * 16, 16)] = jnp.ones((16,), jnp.float32)
        return 0
    lax.fori_loop(0, K // 16, fill, 0)

    def fillz(i, _):
        zeros_v[pl.ds(i * 16, 16)] = jnp.zeros((16,), jnp.float32)
        return 0
    lax.fori_loop(0, RPT // 16, fillz, 0)

    base = sid * RPT
    pltpu.sync_copy(zeros_v, cs_sh.at[pl.ds(base, RPT)])
    pltpu.sync_copy(zeros_v, cd_sh.at[pl.ds(base, RPT)])
    plsc.subcore_barrier()

    def fire(i):
        def go(b):
            j = 8 * i + b
            pltpu.async_copy(ones_v, cs_sh.at[src_v.at[j]], sema, add=True)
            pltpu.async_copy(ones_v, cd_sh.at[dst_v.at[j]], semb, add=True)
        for b in range(8):
            go(b)

    def drain(i):
        def go(b):
            j = 8 * i + b
            pltpu.make_async_copy(ones_v, cs_sh.at[src_v.at[j]], sema).wait()
            pltpu.make_async_copy(ones_v, cd_sh.at[dst_v.at[j]], semb).wait()
        for b in range(8):
            go(b)

    fire(0)

    def body(i, _):
        @pl.when(i + 1 < CH // 8)
        def _():
            fire(i + 1)
        drain(i)
        return 0
    lax.fori_loop(0, CH // 8, body, 0)
    plsc.subcore_barrier()

    pltpu.sync_copy(cs_sh.at[pl.ds(base, RPT)], out_hbm.at[cid, 0, pl.ds(base, RPT)])
    pltpu.sync_copy(cd_sh.at[pl.ds(base, RPT)], out_hbm.at[cid, 1, pl.ds(base, RPT)])


CPT = NW * CH // NS
NBUF = 5
AHEAD = 4


@functools.partial(
    pl.kernel,
    out_type=jax.ShapeDtypeStruct((NPAD, D), jnp.float32),
    mesh=_sc_mesh,
    compiler_params=_sc_params,
    scratch_types=[
        pltpu.VMEM((CPT, K), jnp.int32),
        pltpu.VMEM((CPT, K), jnp.int32),
        pltpu.VMEM((NBUF, K, DH), jnp.float32),
        pltpu.VMEM_SHARED((NPAD, DH), jnp.float32),
        [pltpu.SemaphoreType.DMA] * NBUF,
        [pltpu.SemaphoreType.DMA] * NBUF,
    ],
)
def _agg_kernel(u_hbm, src_hbm, dst_hbm, out_hbm, src_v, dst_v, rows_v,
                acc_sh, gsems, ssems):
    cid = lax.axis_index("c")
    sid = lax.axis_index("s")

    pltpu.make_async_copy(src_hbm.at[cid, pl.ds(sid * CPT, CPT)], src_v,
                          gsems[0]).start()
    pltpu.make_async_copy(dst_hbm.at[pl.ds(sid * CPT, CPT)], dst_v,
                          gsems[1]).start()

    _zero_rows(rows_v.at[0], K, DH)
    for t in range(NCHK):
        base = sid * RPT + t * K
        pltpu.make_async_copy(rows_v.at[0], acc_sh.at[pl.ds(base, K)],
                              gsems[2]).start()
    for t in range(NCHK):
        base = sid * RPT + t * K
        pltpu.make_async_copy(rows_v.at[0], acc_sh.at[pl.ds(base, K)],
                              gsems[2]).wait()
    pltpu.make_async_copy(src_hbm.at[cid, pl.ds(sid * CPT, CPT)], src_v,
                          gsems[0]).wait()
    pltpu.make_async_copy(dst_hbm.at[pl.ds(sid * CPT, CPT)], dst_v,
                          gsems[1]).wait()
    plsc.subcore_barrier()

    def gather(j, b):
        return pltpu.make_async_copy(u_hbm.at[src_v.at[j]], rows_v.at[b],
                                     gsems[b])

    def scatter_wait(j, b):
        pltpu.make_async_copy(rows_v.at[b], acc_sh.at[dst_v.at[j]],
                              ssems[b]).wait()

    for b in range(AHEAD):
        gather(b, b).start()

    LAG = NBUF - AHEAD

    def body(i, _):
        for b in range(NBUF):
            j = NBUF * i + b
            gather(j, b).wait()
            bn = (b + AHEAD) % NBUF

            @pl.when((j >= LAG) & (j + AHEAD < CPT))
            def _():
                scatter_wait(j - LAG, bn)

            @pl.when(j + AHEAD < CPT)
            def _():
                gather(j + AHEAD, bn).start()

            pltpu.async_copy(rows_v.at[b], acc_sh.at[dst_v.at[j]],
                             ssems[b], add=True)
        return 0
    lax.fori_loop(0, CPT // NBUF, body, 0)

    for b in range(NBUF):
        scatter_wait(CPT - NBUF + b, (CPT - NBUF + b) % NBUF)
    plsc.subcore_barrier()

    for t in range(NCHK):
        base = sid * RPT + t * K
        pltpu.make_async_copy(
            acc_sh.at[pl.ds(base, K)],
            out_hbm.at[pl.ds(base, K), pl.ds(cid * DH, DH)], gsems[0]).start()
    for t in range(NCHK):
        base = sid * RPT + t * K
        pltpu.make_async_copy(
            acc_sh.at[pl.ds(base, K)],
            out_hbm.at[pl.ds(base, K), pl.ds(cid * DH, DH)], gsems[0]).wait()


def _mm0_body(x_ref, w_ref, cs_ref, u_ref):
    ns = lax.rsqrt(jnp.maximum(cs_ref[0] + cs_ref[1], 1.0))
    h = x_ref[...] * ns
    u_ref[...] = jnp.dot(h, w_ref[...], preferred_element_type=jnp.float32)


def _finmm_body(agg_ref, b_ref, wn_ref, cs_ref, cd_ref, z_ref, un_ref):
    nd = lax.rsqrt(jnp.maximum(cd_ref[0] + cd_ref[1], 1.0))
    ns = lax.rsqrt(jnp.maximum(cs_ref[0] + cs_ref[1], 1.0))
    z = agg_ref[...] * nd + b_ref[...]
    z_ref[...] = z
    xn = jnp.maximum(z, 0.0) * ns
    un_ref[...] = jnp.dot(xn, wn_ref[...], preferred_element_type=jnp.float32)


_mm0 = pl.pallas_call(
    _mm0_body,
    out_shape=jax.ShapeDtypeStruct((NPAD, D), jnp.float32),
)

_finmm = pl.pallas_call(
    _finmm_body,
    out_shape=(
        jax.ShapeDtypeStruct((NPAD, D), jnp.float32),
        jax.ShapeDtypeStruct((NPAD, D), jnp.float32),
    ),
)


def kernel(features, edge_index, W1, b1, W2, b2):
    src = edge_index[0].astype(jnp.int32)
    dst = edge_index[1].astype(jnp.int32)
    pad_idx = N + (jnp.arange(EPAD - E, dtype=jnp.int32) % (NPAD - N))
    src_p = jnp.concatenate([src, pad_idx]).reshape(NW * CH, K)
    dst_p = jnp.concatenate([dst, pad_idx]).reshape(NW * CH, K)
    feats_p = jnp.pad(features, ((0, NPAD - N), (0, 0)))
    src2_p = jnp.stack([src_p * 2, src_p * 2 + 1])

    cnt = _count_kernel(src_p, dst_p)
    cs = cnt[:, 0, :, None]
    cd = cnt[:, 1, :, None]

    Ws = jnp.stack([W2, W2])
    bs = jnp.stack([b1.reshape(1, D), b2.reshape(1, D)])

    u1 = _mm0(feats_p, W1, cs)

    def layer(l, carry):
        u, _ = carry
        b = lax.dynamic_index_in_dim(bs, l, keepdims=False)
        wn = lax.dynamic_index_in_dim(Ws, l, keepdims=False)
        u_r = u.reshape(2 * NPAD, DH)
        agg = _agg_kernel(u_r, src2_p, dst_p)
        z, un = _finmm(agg, b, wn, cs, cd)
        return (un, z)

    _, z = lax.fori_loop(0, 2, layer, (u1, feats_p))
    return z[:N]
